# initial kernel scaffold (unmeasured)
import jax
import jax.numpy as jnp
from jax import lax
from jax.experimental import pallas as pl
from jax.experimental.pallas import tpu as pltpu


def kernel(
    x,
):
    def body(*refs):
        pass

    out_shape = jax.ShapeDtypeStruct(..., jnp.float32)
    return pl.pallas_call(body, out_shape=out_shape)(...)


# baseline (device time: 793658 ns/iter reference)
import jax
import jax.numpy as jnp
from jax import lax
from jax.experimental import pallas as pl
from jax.experimental.pallas import tpu as pltpu

CHUNK_M = 2048


def kernel(x):
    m_per, n = x.shape
    num_chunks = m_per // CHUNK_M

    def body(x_ref, out_ref, comm_ref, send_sems, recv_sems, credit_sem):
        h = pl.program_id(0)
        my_x = lax.axis_index("x")
        my_y = lax.axis_index("y")
        nbr = (1 - my_x, my_y)
        slot = lax.rem(h, 2)

        @pl.when(h == 0)
        def _():
            barrier_sem = pltpu.get_barrier_semaphore()
            pl.semaphore_signal(
                barrier_sem, inc=1,
                device_id=nbr, device_id_type=pl.DeviceIdType.MESH,
            )
            pl.semaphore_wait(barrier_sem, 1)

        @pl.when(h >= 2)
        def _():
            pl.semaphore_wait(credit_sem, 1)

        rdma = pltpu.make_async_remote_copy(
            src_ref=x_ref,
            dst_ref=comm_ref.at[slot],
            send_sem=send_sems.at[h],
            recv_sem=recv_sems.at[h],
            device_id=nbr,
            device_id_type=pl.DeviceIdType.MESH,
        )
        rdma.start()
        rdma.wait_recv()

        out_ref[...] = x_ref[...] + comm_ref[slot]

        @pl.when(h < num_chunks - 2)
        def _():
            pl.semaphore_signal(
                credit_sem, inc=1,
                device_id=nbr, device_id_type=pl.DeviceIdType.MESH,
            )

        rdma.wait_send()

    return pl.pallas_call(
        body,
        grid=(num_chunks,),
        out_shape=jax.ShapeDtypeStruct((m_per, n), x.dtype),
        in_specs=[
            pl.BlockSpec((CHUNK_M, n), lambda h: (h, 0)),
        ],
        out_specs=pl.BlockSpec((CHUNK_M, n), lambda h: (h, 0)),
        scratch_shapes=[
            pltpu.VMEM((2, CHUNK_M, n), x.dtype),
            pltpu.SemaphoreType.DMA((num_chunks,)),
            pltpu.SemaphoreType.DMA((num_chunks,)),
            pltpu.SemaphoreType.REGULAR,
        ],
        compiler_params=pltpu.CompilerParams(
            collective_id=0,
            dimension_semantics=("arbitrary",),
            vmem_limit_bytes=100 * 1024 * 1024,
        ),
    )(x)


# device time: 481022 ns/iter; 1.6499x vs baseline; 1.6499x over previous
import jax
import jax.numpy as jnp
from jax import lax
from jax.experimental import pallas as pl
from jax.experimental.pallas import tpu as pltpu

CHUNK_M = 1024


def kernel(x):
    m_per, n = x.shape
    half = m_per // 2
    num_chunks = half // CHUNK_M

    def body(
        x_ref, out_ref,
        a_buf, m_buf, s_buf,
        a_send, a_recv, b_send, b_recv,
        m_sem, o_sem, credit_sem,
    ):
        c = pl.program_id(0)
        my_x = lax.axis_index("x")
        my_y = lax.axis_index("y")
        xnbr = (1 - my_x, my_y)
        ynbr = (my_x, 1 - my_y)
        h = lax.rem(my_x + my_y, 2)
        slot = lax.rem(c, 2)

        base_mine = h * half + c * CHUNK_M
        base_send = (1 - h) * half + c * CHUNK_M
        base_recv_b = (1 - h) * half + c * CHUNK_M

        @pl.when(c == 0)
        def _():
            barrier_sem = pltpu.get_barrier_semaphore()
            for nbr in (xnbr, ynbr):
                pl.semaphore_signal(
                    barrier_sem, inc=1,
                    device_id=nbr, device_id_type=pl.DeviceIdType.MESH,
                )
            pl.semaphore_wait(barrier_sem, 2)

        loc = pltpu.make_async_copy(
            x_ref.at[pl.ds(base_mine, CHUNK_M), :], m_buf.at[slot],
            m_sem.at[slot],
        )
        loc.start()

        @pl.when(c >= 2)
        def _():
            pl.semaphore_wait(credit_sem, 1)

        rdma_a = pltpu.make_async_remote_copy(
            src_ref=x_ref.at[pl.ds(base_send, CHUNK_M), :],
            dst_ref=a_buf.at[slot],
            send_sem=a_send.at[c],
            recv_sem=a_recv.at[c],
            device_id=xnbr,
            device_id_type=pl.DeviceIdType.MESH,
        )
        rdma_a.start()

        @pl.when(c >= 2)
        def _():
            pltpu.make_async_remote_copy(
                src_ref=s_buf.at[slot],
                dst_ref=out_ref.at[pl.ds(base_mine, CHUNK_M), :],
                send_sem=b_send.at[c - 2],
                recv_sem=b_recv.at[c - 2],
                device_id=ynbr,
                device_id_type=pl.DeviceIdType.MESH,
            ).wait_send()
            pltpu.make_async_copy(
                s_buf.at[slot],
                out_ref.at[pl.ds(base_mine, CHUNK_M), :],
                o_sem.at[slot],
            ).wait()

        loc.wait()
        rdma_a.wait_recv()
        s_buf[slot] = m_buf[slot] + a_buf[slot]

        @pl.when(c < num_chunks - 2)
        def _():
            pl.semaphore_signal(
                credit_sem, inc=1,
                device_id=xnbr, device_id_type=pl.DeviceIdType.MESH,
            )

        rdma_b = pltpu.make_async_remote_copy(
            src_ref=s_buf.at[slot],
            dst_ref=out_ref.at[pl.ds(base_mine, CHUNK_M), :],
            send_sem=b_send.at[c],
            recv_sem=b_recv.at[c],
            device_id=ynbr,
            device_id_type=pl.DeviceIdType.MESH,
        )
        rdma_b.start()
        pltpu.make_async_copy(
            s_buf.at[slot],
            out_ref.at[pl.ds(base_mine, CHUNK_M), :],
            o_sem.at[slot],
        ).start()

        rdma_a.wait_send()

        @pl.when(c >= 1)
        def _():
            pltpu.make_async_remote_copy(
                src_ref=s_buf.at[1 - slot],
                dst_ref=out_ref.at[
                    pl.ds(base_recv_b - CHUNK_M, CHUNK_M), :
                ],
                send_sem=b_send.at[c - 1],
                recv_sem=b_recv.at[c - 1],
                device_id=ynbr,
                device_id_type=pl.DeviceIdType.MESH,
            ).wait_recv()

        @pl.when(c == num_chunks - 1)
        def _():
            pltpu.make_async_remote_copy(
                src_ref=s_buf.at[slot],
                dst_ref=out_ref.at[pl.ds(base_recv_b, CHUNK_M), :],
                send_sem=b_send.at[c],
                recv_sem=b_recv.at[c],
                device_id=ynbr,
                device_id_type=pl.DeviceIdType.MESH,
            ).wait_recv()
            for back in (1, 0):
                pltpu.make_async_remote_copy(
                    src_ref=s_buf.at[lax.rem(c - back, 2)],
                    dst_ref=out_ref.at[pl.ds(base_mine, CHUNK_M), :],
                    send_sem=b_send.at[c - back],
                    recv_sem=b_recv.at[c - back],
                    device_id=ynbr,
                    device_id_type=pl.DeviceIdType.MESH,
                ).wait_send()
                pltpu.make_async_copy(
                    s_buf.at[lax.rem(c - back, 2)],
                    out_ref.at[pl.ds(base_mine, CHUNK_M), :],
                    o_sem.at[lax.rem(c - back, 2)],
                ).wait()

    return pl.pallas_call(
        body,
        grid=(num_chunks,),
        out_shape=jax.ShapeDtypeStruct((m_per, n), x.dtype),
        in_specs=[pl.BlockSpec(memory_space=pl.ANY)],
        out_specs=pl.BlockSpec(memory_space=pl.ANY),
        scratch_shapes=[
            pltpu.VMEM((2, CHUNK_M, n), x.dtype),
            pltpu.VMEM((2, CHUNK_M, n), x.dtype),
            pltpu.VMEM((2, CHUNK_M, n), x.dtype),
            pltpu.SemaphoreType.DMA((num_chunks,)),
            pltpu.SemaphoreType.DMA((num_chunks,)),
            pltpu.SemaphoreType.DMA((num_chunks,)),
            pltpu.SemaphoreType.DMA((num_chunks,)),
            pltpu.SemaphoreType.DMA((2,)),
            pltpu.SemaphoreType.DMA((2,)),
            pltpu.SemaphoreType.REGULAR,
        ],
        compiler_params=pltpu.CompilerParams(
            collective_id=0,
            dimension_semantics=("arbitrary",),
            vmem_limit_bytes=100 * 1024 * 1024,
        ),
    )(x)


# device time: 456267 ns/iter; 1.7395x vs baseline; 1.0543x over previous
import jax
import jax.numpy as jnp
from jax import lax
from jax.experimental import pallas as pl
from jax.experimental.pallas import tpu as pltpu

CHUNK_M = 1024


def kernel(x):
    m_per, n = x.shape
    half = m_per // 2
    num_chunks = half // CHUNK_M

    def body(
        x_ref, out_ref,
        a_buf, m_buf, s_buf,
        a_send, a_recv, b_send, b_recv,
        m_sem, o_sem, credit_sem,
    ):
        c = pl.program_id(0)
        my_x = lax.axis_index("x")
        my_y = lax.axis_index("y")
        xnbr = (1 - my_x, my_y)
        ynbr = (my_x, 1 - my_y)
        h = lax.rem(my_x + my_y, 2)
        slot = lax.rem(c, 2)

        base_mine = h * half + c * CHUNK_M
        base_send = (1 - h) * half + c * CHUNK_M
        base_recv_b = (1 - h) * half + c * CHUNK_M

        def make_a(j):
            return pltpu.make_async_remote_copy(
                src_ref=x_ref.at[pl.ds((1 - h) * half + j * CHUNK_M, CHUNK_M), :],
                dst_ref=a_buf.at[lax.rem(j, 2)],
                send_sem=a_send.at[j],
                recv_sem=a_recv.at[j],
                device_id=xnbr,
                device_id_type=pl.DeviceIdType.MESH,
            )

        @pl.when(c == 0)
        def _():
            barrier_sem = pltpu.get_barrier_semaphore()
            for nbr in (xnbr, ynbr):
                pl.semaphore_signal(
                    barrier_sem, inc=1,
                    device_id=nbr, device_id_type=pl.DeviceIdType.MESH,
                )
            pl.semaphore_wait(barrier_sem, 2)
            make_a(0).start()
            make_a(1).start()

        loc = pltpu.make_async_copy(
            x_ref.at[pl.ds(base_mine, CHUNK_M), :], m_buf.at[slot],
            m_sem.at[slot],
        )
        loc.start()

        @pl.when(jnp.logical_and(c >= 1, c <= num_chunks - 2))
        def _():
            pl.semaphore_wait(credit_sem, 1)
            make_a(c + 1).start()

        rdma_a = make_a(c)

        @pl.when(c >= 2)
        def _():
            pltpu.make_async_remote_copy(
                src_ref=s_buf.at[slot],
                dst_ref=out_ref.at[pl.ds(base_mine, CHUNK_M), :],
                send_sem=b_send.at[c - 2],
                recv_sem=b_recv.at[c - 2],
                device_id=ynbr,
                device_id_type=pl.DeviceIdType.MESH,
            ).wait_send()
            pltpu.make_async_copy(
                s_buf.at[slot],
                out_ref.at[pl.ds(base_mine, CHUNK_M), :],
                o_sem.at[slot],
            ).wait()

        loc.wait()
        rdma_a.wait_recv()
        s_buf[slot] = m_buf[slot] + a_buf[slot]

        @pl.when(c < num_chunks - 2)
        def _():
            pl.semaphore_signal(
                credit_sem, inc=1,
                device_id=xnbr, device_id_type=pl.DeviceIdType.MESH,
            )

        rdma_b = pltpu.make_async_remote_copy(
            src_ref=s_buf.at[slot],
            dst_ref=out_ref.at[pl.ds(base_mine, CHUNK_M), :],
            send_sem=b_send.at[c],
            recv_sem=b_recv.at[c],
            device_id=ynbr,
            device_id_type=pl.DeviceIdType.MESH,
        )
        rdma_b.start()
        pltpu.make_async_copy(
            s_buf.at[slot],
            out_ref.at[pl.ds(base_mine, CHUNK_M), :],
            o_sem.at[slot],
        ).start()

        rdma_a.wait_send()

        @pl.when(c >= 1)
        def _():
            pltpu.make_async_remote_copy(
                src_ref=s_buf.at[1 - slot],
                dst_ref=out_ref.at[
                    pl.ds(base_recv_b - CHUNK_M, CHUNK_M), :
                ],
                send_sem=b_send.at[c - 1],
                recv_sem=b_recv.at[c - 1],
                device_id=ynbr,
                device_id_type=pl.DeviceIdType.MESH,
            ).wait_recv()

        @pl.when(c == num_chunks - 1)
        def _():
            pltpu.make_async_remote_copy(
                src_ref=s_buf.at[slot],
                dst_ref=out_ref.at[pl.ds(base_recv_b, CHUNK_M), :],
                send_sem=b_send.at[c],
                recv_sem=b_recv.at[c],
                device_id=ynbr,
                device_id_type=pl.DeviceIdType.MESH,
            ).wait_recv()
            for back in (1, 0):
                pltpu.make_async_remote_copy(
                    src_ref=s_buf.at[lax.rem(c - back, 2)],
                    dst_ref=out_ref.at[pl.ds(base_mine, CHUNK_M), :],
                    send_sem=b_send.at[c - back],
                    recv_sem=b_recv.at[c - back],
                    device_id=ynbr,
                    device_id_type=pl.DeviceIdType.MESH,
                ).wait_send()
                pltpu.make_async_copy(
                    s_buf.at[lax.rem(c - back, 2)],
                    out_ref.at[pl.ds(base_mine, CHUNK_M), :],
                    o_sem.at[lax.rem(c - back, 2)],
                ).wait()

    return pl.pallas_call(
        body,
        grid=(num_chunks,),
        out_shape=jax.ShapeDtypeStruct((m_per, n), x.dtype),
        in_specs=[pl.BlockSpec(memory_space=pl.ANY)],
        out_specs=pl.BlockSpec(memory_space=pl.ANY),
        scratch_shapes=[
            pltpu.VMEM((2, CHUNK_M, n), x.dtype),
            pltpu.VMEM((2, CHUNK_M, n), x.dtype),
            pltpu.VMEM((2, CHUNK_M, n), x.dtype),
            pltpu.SemaphoreType.DMA((num_chunks,)),
            pltpu.SemaphoreType.DMA((num_chunks,)),
            pltpu.SemaphoreType.DMA((num_chunks,)),
            pltpu.SemaphoreType.DMA((num_chunks,)),
            pltpu.SemaphoreType.DMA((2,)),
            pltpu.SemaphoreType.DMA((2,)),
            pltpu.SemaphoreType.REGULAR,
        ],
        compiler_params=pltpu.CompilerParams(
            collective_id=0,
            dimension_semantics=("arbitrary",),
            vmem_limit_bytes=100 * 1024 * 1024,
        ),
    )(x)


# device time: 433775 ns/iter; 1.8297x vs baseline; 1.0519x over previous
import jax
import jax.numpy as jnp
from jax import lax
from jax.experimental import pallas as pl
from jax.experimental.pallas import tpu as pltpu

CHUNK_M = 512


def kernel(x):
    m_per, n = x.shape
    half = m_per // 2
    num_chunks = half // CHUNK_M

    def body(
        x_ref, out_ref,
        a_buf, m_buf, s_buf,
        a_send, a_recv, b_send, b_recv,
        m_sem, o_sem, credit_sem,
    ):
        c = pl.program_id(0)
        my_x = lax.axis_index("x")
        my_y = lax.axis_index("y")
        xnbr = (1 - my_x, my_y)
        ynbr = (my_x, 1 - my_y)
        h = lax.rem(my_x + my_y, 2)
        slot = lax.rem(c, 2)
        aslot = lax.rem(c, 3)

        base_mine = h * half + c * CHUNK_M
        base_send = (1 - h) * half + c * CHUNK_M
        base_recv_b = (1 - h) * half + c * CHUNK_M

        def make_a(j):
            return pltpu.make_async_remote_copy(
                src_ref=x_ref.at[pl.ds((1 - h) * half + j * CHUNK_M, CHUNK_M), :],
                dst_ref=a_buf.at[lax.rem(j, 3)],
                send_sem=a_send.at[j],
                recv_sem=a_recv.at[j],
                device_id=xnbr,
                device_id_type=pl.DeviceIdType.MESH,
            )

        @pl.when(c == 0)
        def _():
            barrier_sem = pltpu.get_barrier_semaphore()
            for nbr in (xnbr, ynbr):
                pl.semaphore_signal(
                    barrier_sem, inc=1,
                    device_id=nbr, device_id_type=pl.DeviceIdType.MESH,
                )
            pl.semaphore_wait(barrier_sem, 2)
            make_a(0).start()
            make_a(1).start()
            make_a(2).start()

        loc = pltpu.make_async_copy(
            x_ref.at[pl.ds(base_mine, CHUNK_M), :], m_buf.at[slot],
            m_sem.at[slot],
        )
        loc.start()

        @pl.when(jnp.logical_and(c >= 1, c <= num_chunks - 3))
        def _():
            pl.semaphore_wait(credit_sem, 1)
            make_a(c + 2).start()

        rdma_a = make_a(c)

        @pl.when(c >= 2)
        def _():
            pltpu.make_async_remote_copy(
                src_ref=s_buf.at[slot],
                dst_ref=out_ref.at[pl.ds(base_mine, CHUNK_M), :],
                send_sem=b_send.at[c - 2],
                recv_sem=b_recv.at[c - 2],
                device_id=ynbr,
                device_id_type=pl.DeviceIdType.MESH,
            ).wait_send()
            pltpu.make_async_copy(
                s_buf.at[slot],
                out_ref.at[pl.ds(base_mine, CHUNK_M), :],
                o_sem.at[slot],
            ).wait()

        loc.wait()
        rdma_a.wait_recv()
        s_buf[slot] = m_buf[slot] + a_buf[aslot]

        @pl.when(c < num_chunks - 3)
        def _():
            pl.semaphore_signal(
                credit_sem, inc=1,
                device_id=xnbr, device_id_type=pl.DeviceIdType.MESH,
            )

        rdma_b = pltpu.make_async_remote_copy(
            src_ref=s_buf.at[slot],
            dst_ref=out_ref.at[pl.ds(base_mine, CHUNK_M), :],
            send_sem=b_send.at[c],
            recv_sem=b_recv.at[c],
            device_id=ynbr,
            device_id_type=pl.DeviceIdType.MESH,
        )
        rdma_b.start()
        pltpu.make_async_copy(
            s_buf.at[slot],
            out_ref.at[pl.ds(base_mine, CHUNK_M), :],
            o_sem.at[slot],
        ).start()

        rdma_a.wait_send()

        @pl.when(c >= 1)
        def _():
            pltpu.make_async_remote_copy(
                src_ref=s_buf.at[1 - slot],
                dst_ref=out_ref.at[
                    pl.ds(base_recv_b - CHUNK_M, CHUNK_M), :
                ],
                send_sem=b_send.at[c - 1],
                recv_sem=b_recv.at[c - 1],
                device_id=ynbr,
                device_id_type=pl.DeviceIdType.MESH,
            ).wait_recv()

        @pl.when(c == num_chunks - 1)
        def _():
            pltpu.make_async_remote_copy(
                src_ref=s_buf.at[slot],
                dst_ref=out_ref.at[pl.ds(base_recv_b, CHUNK_M), :],
                send_sem=b_send.at[c],
                recv_sem=b_recv.at[c],
                device_id=ynbr,
                device_id_type=pl.DeviceIdType.MESH,
            ).wait_recv()
            for back in (1, 0):
                pltpu.make_async_remote_copy(
                    src_ref=s_buf.at[lax.rem(c - back, 2)],
                    dst_ref=out_ref.at[pl.ds(base_mine, CHUNK_M), :],
                    send_sem=b_send.at[c - back],
                    recv_sem=b_recv.at[c - back],
                    device_id=ynbr,
                    device_id_type=pl.DeviceIdType.MESH,
                ).wait_send()
                pltpu.make_async_copy(
                    s_buf.at[lax.rem(c - back, 2)],
                    out_ref.at[pl.ds(base_mine, CHUNK_M), :],
                    o_sem.at[lax.rem(c - back, 2)],
                ).wait()

    return pl.pallas_call(
        body,
        grid=(num_chunks,),
        out_shape=jax.ShapeDtypeStruct((m_per, n), x.dtype),
        in_specs=[pl.BlockSpec(memory_space=pl.ANY)],
        out_specs=pl.BlockSpec(memory_space=pl.ANY),
        scratch_shapes=[
            pltpu.VMEM((3, CHUNK_M, n), x.dtype),
            pltpu.VMEM((2, CHUNK_M, n), x.dtype),
            pltpu.VMEM((2, CHUNK_M, n), x.dtype),
            pltpu.SemaphoreType.DMA((num_chunks,)),
            pltpu.SemaphoreType.DMA((num_chunks,)),
            pltpu.SemaphoreType.DMA((num_chunks,)),
            pltpu.SemaphoreType.DMA((num_chunks,)),
            pltpu.SemaphoreType.DMA((2,)),
            pltpu.SemaphoreType.DMA((2,)),
            pltpu.SemaphoreType.REGULAR,
        ],
        compiler_params=pltpu.CompilerParams(
            collective_id=0,
            dimension_semantics=("arbitrary",),
            vmem_limit_bytes=100 * 1024 * 1024,
        ),
    )(x)
